# baseline (speedup 1.0000x reference)
import functools

import jax
import jax.numpy as jnp
from jax import lax
from jax.experimental import pallas as pl
from jax.experimental.pallas import tpu as pltpu
from jax.experimental.pallas import tpu_sc as plsc

B = 4096
D = 1024
FF = 2048
E = 8
K = 2
OUT = 1024
TB = 256
BS = 256
NB = 40
NTOT = NB * BS
NW = 32
TPW = B // NW
NA = B * K
NT = B // TB

_SC_MESH = dict(core_axis_name="c", subcore_axis_name="s",
                num_cores=2, num_subcores=16)
_SC_PARAMS = pltpu.CompilerParams(needs_layout_passes=False)


def _gate_body(x_ref, m_ref, wg_ref, bg_ref,
               xm_ref, eidx_ref, gw_ref, base_ref, blk_ref, hist_ref):
    t = pl.program_id(0)
    logits = jnp.dot(x_ref[...], wg_ref[...],
                     preferred_element_type=jnp.float32) + bg_ref[...]
    iota = lax.broadcasted_iota(jnp.int32, (TB, E), 1)
    m1 = jnp.max(logits, axis=1, keepdims=True)
    i1 = jnp.min(jnp.where(logits == m1, iota, E + 1), axis=1, keepdims=True)
    sel1 = iota == i1
    neg = jnp.where(sel1, -jnp.inf, logits)
    m2 = jnp.max(neg, axis=1, keepdims=True)
    i2 = jnp.min(jnp.where(neg == m2, iota, E + 1), axis=1, keepdims=True)
    eb = jnp.exp(m2 - m1)
    p1 = 1.0 / (1.0 + eb)
    p2 = eb / (1.0 + eb)
    eidx_ref[...] = jnp.concatenate([i1, i2], axis=1)
    gw_ref[...] = jnp.concatenate([p1, p2], axis=1)
    xm_ref[...] = x_ref[...] * m_ref[...]

    lane16 = lax.broadcasted_iota(jnp.int32, (TB, 16), 1)
    onehot = ((i1 == lane16).astype(jnp.float32)
              + (i2 == lane16).astype(jnp.float32))
    c0 = jnp.sum(onehot[:TPW], axis=0, keepdims=True)
    c1 = jnp.sum(onehot[TPW:], axis=0, keepdims=True)
    hist_ref[t] = jnp.concatenate([c0, c1], axis=0)

    @pl.when(t == NT - 1)
    def _plan():
        cnt = hist_ref[...].reshape(NW, 16)
        wi = lax.broadcasted_iota(jnp.int32, (NW, NW), 0)
        wj = lax.broadcasted_iota(jnp.int32, (NW, NW), 1)
        lt = (wj < wi).astype(jnp.float32)
        s = jnp.dot(lt, cnt, preferred_element_type=jnp.float32)
        ctot = jnp.sum(cnt, axis=0, keepdims=True).astype(jnp.int32)
        nblk = (ctot + (BS - 1)) // BS
        li = lax.broadcasted_iota(jnp.int32, (16, 16), 0)
        lj = lax.broadcasted_iota(jnp.int32, (16, 16), 1)
        le = (li <= lj).astype(jnp.float32)
        endb = jnp.dot(nblk.astype(jnp.float32), le,
                       preferred_element_type=jnp.float32).astype(jnp.int32)
        startb = endb - nblk
        base_ref[...] = startb * BS + s.astype(jnp.int32)
        jv = lax.broadcasted_iota(jnp.int32, (1, 128), 1)
        acc = jnp.zeros((1, 128), jnp.int32)
        l16 = lax.broadcasted_iota(jnp.int32, (1, 16), 1)
        for e in range(E):
            end_e = jnp.sum(jnp.where(l16 == e, endb, 0))
            acc = acc + (jv >= end_e).astype(jnp.int32)
        nb_used = jnp.sum(jnp.where(l16 == E - 1, endb, 0))
        blk_ref[...] = jnp.where(jv == 48, nb_used, jnp.minimum(acc, E - 1))


def _gate(x, mask, Wg, bg):
    return pl.pallas_call(
        _gate_body,
        grid=(NT,),
        in_specs=[
            pl.BlockSpec((TB, D), lambda t: (t, 0)),
            pl.BlockSpec((TB, D), lambda t: (t, 0)),
            pl.BlockSpec((D, E), lambda t: (0, 0)),
            pl.BlockSpec((1, E), lambda t: (0, 0)),
        ],
        out_specs=[
            pl.BlockSpec((TB, D), lambda t: (t, 0)),
            pl.BlockSpec((TB, K), lambda t: (t, 0)),
            pl.BlockSpec((TB, K), lambda t: (t, 0)),
            pl.BlockSpec((NW, 16), lambda t: (0, 0)),
            pl.BlockSpec((1, 128), lambda t: (0, 0)),
        ],
        out_shape=[
            jax.ShapeDtypeStruct((B, D), jnp.float32),
            jax.ShapeDtypeStruct((B, K), jnp.int32),
            jax.ShapeDtypeStruct((B, K), jnp.float32),
            jax.ShapeDtypeStruct((NW, 16), jnp.int32),
            jax.ShapeDtypeStruct((1, 128), jnp.int32),
        ],
        scratch_shapes=[pltpu.VMEM((NT, 2, 16), jnp.float32)],
        compiler_params=pltpu.CompilerParams(
            dimension_semantics=("arbitrary",),
        ),
    )(x, mask, Wg, bg.reshape(1, E))


def _scatter_body(xm_hbm, ef_hbm, basearr_hbm, xs_hbm, pos_hbm,
                  e_v, base_v, dpos_v, rows_a, rows_b,
                  sin_a, sin_b, sout_a, sout_b):
    wid = lax.axis_index("s") * 2 + lax.axis_index("c")
    base = wid * TPW
    lane = lax.broadcasted_iota(jnp.int32, (16,), 0)

    pltpu.sync_copy(ef_hbm.at[pl.ds(base * K, TPW * K)], e_v)
    pltpu.sync_copy(basearr_hbm.at[pl.ds(wid * 16, 16)], base_v)
    base_row = base_v[...]

    run = [jnp.int32(0)] * E
    for v in range(TPW * K // 16):
        ev = e_v[pl.ds(16 * v, 16)]
        dest = jnp.zeros((16,), jnp.int32)
        for e in range(E):
            mi = (ev == e).astype(jnp.int32)
            pref = plsc.cumsum(mi) - mi
            base_e = jnp.sum(jnp.where(lane == e, base_row, 0))
            dest = dest + mi * (base_e + run[e] + pref)
            run[e] = run[e] + jnp.sum(mi)
        dpos_v[pl.ds(16 * v, 16)] = dest

    pltpu.sync_copy(dpos_v, pos_hbm.at[pl.ds(base * K, TPW * K)])

    nc = TPW // 16
    rows = [rows_a, rows_b]
    sin = [sin_a, sin_b]
    sout = [sout_a, sout_b]
    cpin = [None, None]
    cpout = [None, None]

    def _issue_load(c):
        b = c % 2
        cpin[b] = pltpu.async_copy(
            xm_hbm.at[pl.ds(base + 16 * c, 16)], rows[b], sin[b])

    _issue_load(0)
    for c in range(nc):
        b = c % 2
        if c + 1 < nc:
            nb = (c + 1) % 2
            if cpout[nb] is not None:
                cpout[nb][0].wait()
                cpout[nb][1].wait()
                cpout[nb] = None
            _issue_load(c + 1)
        cpin[b].wait()
        i0 = plsc.load_gather(dpos_v, [32 * c + 2 * lane])
        i1 = plsc.load_gather(dpos_v, [32 * c + 2 * lane + 1])
        cpout[b] = (pltpu.async_copy(rows[b], xs_hbm.at[i0], sout[b]),
                    pltpu.async_copy(rows[b], xs_hbm.at[i1], sout[b]))
    for b in range(2):
        if cpout[b] is not None:
            cpout[b][0].wait()
            cpout[b][1].wait()


@functools.cache
def _scatter_kernel():
    return pl.kernel(
        _scatter_body,
        out_type=[
            jax.ShapeDtypeStruct((NTOT, D), jnp.float32),
            jax.ShapeDtypeStruct((NA,), jnp.int32),
        ],
        mesh=plsc.VectorSubcoreMesh(**_SC_MESH),
        scratch_types=[
            pltpu.VMEM((TPW * K,), jnp.int32),
            pltpu.VMEM((16,), jnp.int32),
            pltpu.VMEM((TPW * K,), jnp.int32),
            pltpu.VMEM((16, D), jnp.float32),
            pltpu.VMEM((16, D), jnp.float32),
            pltpu.SemaphoreType.DMA,
            pltpu.SemaphoreType.DMA,
            pltpu.SemaphoreType.DMA,
            pltpu.SemaphoreType.DMA,
        ],
        compiler_params=_SC_PARAMS,
    )


def _gmm_body(s_ref, xs_ref, w1_ref, b1_ref, w2_ref, b2_ref, ys_ref):
    j = pl.program_id(0)

    @pl.when(j < s_ref[48])
    def _():
        h = jnp.maximum(
            jnp.dot(xs_ref[...], w1_ref[0],
                    preferred_element_type=jnp.float32) + b1_ref[0], 0.0)
        ys_ref[...] = jnp.dot(h, w2_ref[0],
                              preferred_element_type=jnp.float32) + b2_ref[0]


def _gmm(blk, xs, W1, b1, W2, b2):
    grid_spec = pltpu.PrefetchScalarGridSpec(
        num_scalar_prefetch=1,
        grid=(NB,),
        in_specs=[
            pl.BlockSpec((BS, D), lambda j, s: (j, 0)),
            pl.BlockSpec((1, D, FF), lambda j, s: (s[j], 0, 0)),
            pl.BlockSpec((1, 1, FF), lambda j, s: (s[j], 0, 0)),
            pl.BlockSpec((1, FF, OUT), lambda j, s: (s[j], 0, 0)),
            pl.BlockSpec((1, 1, OUT), lambda j, s: (s[j], 0, 0)),
        ],
        out_specs=pl.BlockSpec((BS, OUT), lambda j, s: (j, 0)),
    )
    return pl.pallas_call(
        _gmm_body,
        grid_spec=grid_spec,
        out_shape=jax.ShapeDtypeStruct((NTOT, OUT), jnp.float32),
        compiler_params=pltpu.CompilerParams(
            dimension_semantics=("arbitrary",),
        ),
    )(blk, xs, W1, b1.reshape(E, 1, FF), W2, b2.reshape(E, 1, OUT))


def _combine_body(ys_hbm, pos_hbm, gw_hbm, out_hbm,
                  pos_v, w_v, r_a, r_b, o_v, sem_a, sem_b):
    wid = lax.axis_index("s") * 2 + lax.axis_index("c")
    base = wid * TPW

    pltpu.sync_copy(pos_hbm.at[pl.ds(base * K, TPW * K)], pos_v)
    pltpu.sync_copy(gw_hbm.at[pl.ds(base * K, TPW * K)], w_v)

    CH = 32
    nc = TPW * K // CH
    r = [r_a, r_b]
    sems = [sem_a, sem_b]

    def _issue(c):
        b = c % 2
        return pltpu.async_copy(
            ys_hbm.at[pos_v.at[pl.ds(CH * c, CH)]], r[b], sems[b])

    cps = _issue(0)
    for c in range(nc):
        b = c % 2
        nxt = _issue(c + 1) if c + 1 < nc else None
        cps.wait()

        def row_body(t, carry):
            w0 = plsc.load_gather(w_v, [jnp.zeros((16,), jnp.int32)
                                        + (CH * c + 2 * t)])
            w1 = plsc.load_gather(w_v, [jnp.zeros((16,), jnp.int32)
                                        + (CH * c + 2 * t + 1)])

            def col_body(k8, carry2):
                for u in range(8):
                    kk = 8 * k8 + u
                    a = r[b][2 * t, pl.ds(16 * kk, 16)]
                    bb = r[b][2 * t + 1, pl.ds(16 * kk, 16)]
                    o_v[t, pl.ds(16 * kk, 16)] = w0 * a + w1 * bb
                return carry2

            return lax.fori_loop(0, OUT // 128, col_body, carry)

        lax.fori_loop(0, CH // 2, row_body, 0)
        pltpu.sync_copy(o_v, out_hbm.at[pl.ds(base + (CH // 2) * c, CH // 2)])
        cps = nxt


@functools.cache
def _combine_kernel():
    return pl.kernel(
        _combine_body,
        out_type=jax.ShapeDtypeStruct((B, OUT), jnp.float32),
        mesh=plsc.VectorSubcoreMesh(**_SC_MESH),
        scratch_types=[
            pltpu.VMEM((TPW * K,), jnp.int32),
            pltpu.VMEM((TPW * K,), jnp.float32),
            pltpu.VMEM((32, OUT), jnp.float32),
            pltpu.VMEM((32, OUT), jnp.float32),
            pltpu.VMEM((16, OUT), jnp.float32),
            pltpu.SemaphoreType.DMA,
            pltpu.SemaphoreType.DMA,
        ],
        compiler_params=_SC_PARAMS,
    )


@jax.jit
def kernel(insample_y, insample_mask, Wg, bg, W1, b1, W2, b2):
    xm, eidx, gw, basearr, blk = _gate(insample_y, insample_mask, Wg, bg)
    ef = eidx.reshape(NA)
    xs, pos = _scatter_kernel()(xm, ef, basearr.reshape(NW * 16))
    ys = _gmm(blk.reshape(128), xs, W1, b1, W2, b2)
    return _combine_kernel()(ys, pos, gw.reshape(NA))


# speedup vs baseline: 1.1259x; 1.1259x over previous
import functools

import jax
import jax.numpy as jnp
from jax import lax
from jax.experimental import pallas as pl
from jax.experimental.pallas import tpu as pltpu
from jax.experimental.pallas import tpu_sc as plsc

B = 4096
D = 1024
FF = 2048
E = 8
K = 2
OUT = 1024
TB = 256
BS = 256
NB = 40
NTOT = NB * BS
NW = 32
TPW = B // NW
NA = B * K
NT = B // TB

_SC_MESH = dict(core_axis_name="c", subcore_axis_name="s",
                num_cores=2, num_subcores=16)
_SC_PARAMS = pltpu.CompilerParams(needs_layout_passes=False)


def _gate_body(x_ref, m_ref, wg_ref, bg_ref,
               xm_ref, eidx_ref, gw_ref, base_ref, blk_ref, hist_ref):
    t = pl.program_id(0)
    logits = jnp.dot(x_ref[...], wg_ref[...],
                     preferred_element_type=jnp.float32) + bg_ref[...]
    iota = lax.broadcasted_iota(jnp.int32, (TB, E), 1)
    m1 = jnp.max(logits, axis=1, keepdims=True)
    i1 = jnp.min(jnp.where(logits == m1, iota, E + 1), axis=1, keepdims=True)
    sel1 = iota == i1
    neg = jnp.where(sel1, -jnp.inf, logits)
    m2 = jnp.max(neg, axis=1, keepdims=True)
    i2 = jnp.min(jnp.where(neg == m2, iota, E + 1), axis=1, keepdims=True)
    eb = jnp.exp(m2 - m1)
    p1 = 1.0 / (1.0 + eb)
    p2 = eb / (1.0 + eb)
    eidx_ref[...] = jnp.concatenate([i1, i2], axis=1)
    gw_ref[...] = jnp.concatenate([p1, p2], axis=1)
    xm_ref[...] = x_ref[...] * m_ref[...]

    lane16 = lax.broadcasted_iota(jnp.int32, (TB, 16), 1)
    onehot = ((i1 == lane16).astype(jnp.float32)
              + (i2 == lane16).astype(jnp.float32))
    c0 = jnp.sum(onehot[:TPW], axis=0, keepdims=True)
    c1 = jnp.sum(onehot[TPW:], axis=0, keepdims=True)
    hist_ref[t] = jnp.concatenate([c0, c1], axis=0)

    @pl.when(t == NT - 1)
    def _plan():
        cnt = hist_ref[...].reshape(NW, 16)
        wi = lax.broadcasted_iota(jnp.int32, (NW, NW), 0)
        wj = lax.broadcasted_iota(jnp.int32, (NW, NW), 1)
        lt = (wj < wi).astype(jnp.float32)
        s = jnp.dot(lt, cnt, preferred_element_type=jnp.float32)
        ctot = jnp.sum(cnt, axis=0, keepdims=True).astype(jnp.int32)
        nblk = (ctot + (BS - 1)) // BS
        li = lax.broadcasted_iota(jnp.int32, (16, 16), 0)
        lj = lax.broadcasted_iota(jnp.int32, (16, 16), 1)
        le = (li <= lj).astype(jnp.float32)
        endb = jnp.dot(nblk.astype(jnp.float32), le,
                       preferred_element_type=jnp.float32).astype(jnp.int32)
        startb = endb - nblk
        base_ref[...] = startb * BS + s.astype(jnp.int32)
        jv = lax.broadcasted_iota(jnp.int32, (1, 128), 1)
        acc = jnp.zeros((1, 128), jnp.int32)
        l16 = lax.broadcasted_iota(jnp.int32, (1, 16), 1)
        for e in range(E):
            end_e = jnp.sum(jnp.where(l16 == e, endb, 0))
            acc = acc + (jv >= end_e).astype(jnp.int32)
        nb_used = jnp.sum(jnp.where(l16 == E - 1, endb, 0))
        blk_ref[...] = jnp.where(jv == 48, nb_used, jnp.minimum(acc, E - 1))


def _gate(x, mask, Wg, bg):
    return pl.pallas_call(
        _gate_body,
        grid=(NT,),
        in_specs=[
            pl.BlockSpec((TB, D), lambda t: (t, 0)),
            pl.BlockSpec((TB, D), lambda t: (t, 0)),
            pl.BlockSpec((D, E), lambda t: (0, 0)),
            pl.BlockSpec((1, E), lambda t: (0, 0)),
        ],
        out_specs=[
            pl.BlockSpec((TB, D), lambda t: (t, 0)),
            pl.BlockSpec((TB, K), lambda t: (t, 0)),
            pl.BlockSpec((TB, K), lambda t: (t, 0)),
            pl.BlockSpec((NW, 16), lambda t: (0, 0)),
            pl.BlockSpec((1, 128), lambda t: (0, 0)),
        ],
        out_shape=[
            jax.ShapeDtypeStruct((B, D), jnp.float32),
            jax.ShapeDtypeStruct((B, K), jnp.int32),
            jax.ShapeDtypeStruct((B, K), jnp.float32),
            jax.ShapeDtypeStruct((NW, 16), jnp.int32),
            jax.ShapeDtypeStruct((1, 128), jnp.int32),
        ],
        scratch_shapes=[pltpu.VMEM((NT, 2, 16), jnp.float32)],
        compiler_params=pltpu.CompilerParams(
            dimension_semantics=("arbitrary",),
        ),
    )(x, mask, Wg, bg.reshape(1, E))


def _scatter_body(xm_hbm, ef_hbm, basearr_hbm, xs_hbm, pos_hbm,
                  e_v, base_v, dpos_v, rows_a, rows_b,
                  i0r_a, i1r_a, i0r_b, i1r_b,
                  sin_a, sin_b, sout_a, sout_b):
    wid = lax.axis_index("s") * 2 + lax.axis_index("c")
    base = wid * TPW
    lane = lax.broadcasted_iota(jnp.int32, (16,), 0)

    pltpu.sync_copy(ef_hbm.at[pl.ds(base * K, TPW * K)], e_v)
    pltpu.sync_copy(basearr_hbm.at[pl.ds(wid * 16, 16)], base_v)
    base_row = base_v[...]

    run = [jnp.int32(0)] * E
    for v in range(TPW * K // 16):
        ev = e_v[pl.ds(16 * v, 16)]
        dest = jnp.zeros((16,), jnp.int32)
        for e in range(E):
            mi = (ev == e).astype(jnp.int32)
            pref = plsc.cumsum(mi) - mi
            base_e = jnp.sum(jnp.where(lane == e, base_row, 0))
            dest = dest + mi * (base_e + run[e] + pref)
            run[e] = run[e] + jnp.sum(mi)
        dpos_v[pl.ds(16 * v, 16)] = dest

    pltpu.sync_copy(dpos_v, pos_hbm.at[pl.ds(base * K, TPW * K)])

    nc = TPW // 16
    rows = [rows_a, rows_b]
    sin = [sin_a, sin_b]
    sout = [sout_a, sout_b]
    cpin = [None, None]
    cpout = [None, None]

    def _issue_load(c):
        b = c % 2
        cpin[b] = pltpu.async_copy(
            xm_hbm.at[pl.ds(base + 16 * c, 16)], rows[b], sin[b])

    i0r = [i0r_a, i0r_b]
    i1r = [i1r_a, i1r_b]
    _issue_load(0)
    for c in range(nc):
        b = c % 2
        if c + 1 < nc:
            nb = (c + 1) % 2
            if cpout[nb] is not None:
                cpout[nb][0].wait()
                cpout[nb][1].wait()
                cpout[nb] = None
            _issue_load(c + 1)
        cpin[b].wait()
        i0r[b][...] = plsc.load_gather(dpos_v, [32 * c + 2 * lane])
        i1r[b][...] = plsc.load_gather(dpos_v, [32 * c + 2 * lane + 1])
        cpout[b] = (pltpu.async_copy(rows[b], xs_hbm.at[i0r[b]], sout[b]),
                    pltpu.async_copy(rows[b], xs_hbm.at[i1r[b]], sout[b]))
    for b in range(2):
        if cpout[b] is not None:
            cpout[b][0].wait()
            cpout[b][1].wait()


@functools.cache
def _scatter_kernel():
    return pl.kernel(
        _scatter_body,
        out_type=[
            jax.ShapeDtypeStruct((NTOT, D), jnp.float32),
            jax.ShapeDtypeStruct((NA,), jnp.int32),
        ],
        mesh=plsc.VectorSubcoreMesh(**_SC_MESH),
        scratch_types=[
            pltpu.VMEM((TPW * K,), jnp.int32),
            pltpu.VMEM((16,), jnp.int32),
            pltpu.VMEM((TPW * K,), jnp.int32),
            pltpu.VMEM((16, D), jnp.float32),
            pltpu.VMEM((16, D), jnp.float32),
            pltpu.VMEM((16,), jnp.int32),
            pltpu.VMEM((16,), jnp.int32),
            pltpu.VMEM((16,), jnp.int32),
            pltpu.VMEM((16,), jnp.int32),
            pltpu.SemaphoreType.DMA,
            pltpu.SemaphoreType.DMA,
            pltpu.SemaphoreType.DMA,
            pltpu.SemaphoreType.DMA,
        ],
        compiler_params=_SC_PARAMS,
    )


def _gmm_body(s_ref, xs_ref, w1_ref, b1_ref, w2_ref, b2_ref, ys_ref):
    j = pl.program_id(0)

    @pl.when(j < s_ref[48])
    def _():
        h = jnp.maximum(
            jnp.dot(xs_ref[...], w1_ref[0],
                    preferred_element_type=jnp.float32) + b1_ref[0], 0.0)
        ys_ref[...] = jnp.dot(h, w2_ref[0],
                              preferred_element_type=jnp.float32) + b2_ref[0]


def _gmm(blk, xs, W1, b1, W2, b2):
    grid_spec = pltpu.PrefetchScalarGridSpec(
        num_scalar_prefetch=1,
        grid=(NB,),
        in_specs=[
            pl.BlockSpec((BS, D), lambda j, s: (j, 0)),
            pl.BlockSpec((1, D, FF), lambda j, s: (s[j], 0, 0)),
            pl.BlockSpec((1, 1, FF), lambda j, s: (s[j], 0, 0)),
            pl.BlockSpec((1, FF, OUT), lambda j, s: (s[j], 0, 0)),
            pl.BlockSpec((1, 1, OUT), lambda j, s: (s[j], 0, 0)),
        ],
        out_specs=pl.BlockSpec((BS, OUT), lambda j, s: (j, 0)),
    )
    return pl.pallas_call(
        _gmm_body,
        grid_spec=grid_spec,
        out_shape=jax.ShapeDtypeStruct((NTOT, OUT), jnp.float32),
        compiler_params=pltpu.CompilerParams(
            dimension_semantics=("arbitrary",),
        ),
    )(blk, xs, W1, b1.reshape(E, 1, FF), W2, b2.reshape(E, 1, OUT))


def _combine_body(ys_hbm, pos_hbm, gw_hbm, out_hbm,
                  pos_v, w_v, r0_a, r1_a, r0_b, r1_b, o_v,
                  p0r_a, p1r_a, p0r_b, p1r_b, sem_a, sem_b):
    wid = lax.axis_index("s") * 2 + lax.axis_index("c")
    base = wid * TPW

    pltpu.sync_copy(pos_hbm.at[pl.ds(base * K, TPW * K)], pos_v)
    pltpu.sync_copy(gw_hbm.at[pl.ds(base * K, TPW * K)], w_v)

    lane = lax.broadcasted_iota(jnp.int32, (16,), 0)
    nc = TPW // 16
    r0 = [r0_a, r0_b]
    r1 = [r1_a, r1_b]
    sems = [sem_a, sem_b]

    p0r = [p0r_a, p0r_b]
    p1r = [p1r_a, p1r_b]

    def _issue(c):
        b = c % 2
        p0r[b][...] = plsc.load_gather(pos_v, [32 * c + 2 * lane])
        p1r[b][...] = plsc.load_gather(pos_v, [32 * c + 2 * lane + 1])
        return (pltpu.async_copy(ys_hbm.at[p0r[b]], r0[b], sems[b]),
                pltpu.async_copy(ys_hbm.at[p1r[b]], r1[b], sems[b]))

    cps = _issue(0)
    for c in range(nc):
        b = c % 2
        nxt = _issue(c + 1) if c + 1 < nc else None
        cps[0].wait()
        cps[1].wait()

        def row_body(t, carry):
            w0 = plsc.load_gather(w_v, [jnp.zeros((16,), jnp.int32)
                                        + (32 * c + 2 * t)])
            w1 = plsc.load_gather(w_v, [jnp.zeros((16,), jnp.int32)
                                        + (32 * c + 2 * t + 1)])

            def col_body(k, carry2):
                a = r0[b][t, pl.ds(16 * k, 16)]
                bb = r1[b][t, pl.ds(16 * k, 16)]
                o_v[t, pl.ds(16 * k, 16)] = w0 * a + w1 * bb
                return carry2

            return lax.fori_loop(0, OUT // 16, col_body, carry)

        lax.fori_loop(0, 16, row_body, 0)
        pltpu.sync_copy(o_v, out_hbm.at[pl.ds(base + 16 * c, 16)])
        cps = nxt


@functools.cache
def _combine_kernel():
    return pl.kernel(
        _combine_body,
        out_type=jax.ShapeDtypeStruct((B, OUT), jnp.float32),
        mesh=plsc.VectorSubcoreMesh(**_SC_MESH),
        scratch_types=[
            pltpu.VMEM((TPW * K,), jnp.int32),
            pltpu.VMEM((TPW * K,), jnp.float32),
            pltpu.VMEM((16, OUT), jnp.float32),
            pltpu.VMEM((16, OUT), jnp.float32),
            pltpu.VMEM((16, OUT), jnp.float32),
            pltpu.VMEM((16, OUT), jnp.float32),
            pltpu.VMEM((16, OUT), jnp.float32),
            pltpu.VMEM((16,), jnp.int32),
            pltpu.VMEM((16,), jnp.int32),
            pltpu.VMEM((16,), jnp.int32),
            pltpu.VMEM((16,), jnp.int32),
            pltpu.SemaphoreType.DMA,
            pltpu.SemaphoreType.DMA,
        ],
        compiler_params=_SC_PARAMS,
    )


@jax.jit
def kernel(insample_y, insample_mask, Wg, bg, W1, b1, W2, b2):
    xm, eidx, gw, basearr, blk = _gate(insample_y, insample_mask, Wg, bg)
    ef = eidx.reshape(NA)
    xs, pos = _scatter_kernel()(xm, ef, basearr.reshape(NW * 16))
    ys = _gmm(blk.reshape(128), xs, W1, b1, W2, b2)
    return _combine_kernel()(ys, pos, gw.reshape(NA))
